# baseline (speedup 1.0000x reference)
import functools

import jax
import jax.numpy as jnp
from jax import lax
from jax.experimental import pallas as pl
from jax.experimental.pallas import tpu as pltpu
from jax.experimental.pallas import tpu_sc as plsc

N = 2_000_000
CHUNK = 8_000
NCHUNKS = N // CHUNK
VECS = CHUNK // 16
NW = 32
JMAX = -(-NCHUNKS // NW)
NBINS = 10
LANES = 16
BSLOTS = 16
SEG = LANES * BSLOTS
ACCLEN = 3 * SEG

_B9 = 0.9


def _tec_body(conf_hbm, pred_hbm, targ_hbm, out_hbm, conf_v, pred_v, targ_v, acc_v):
    wid = lax.axis_index("s") * 2 + lax.axis_index("c")

    zeros = jnp.zeros((LANES,), jnp.float32)
    for i in range(ACCLEN // LANES):
        acc_v[pl.ds(i * LANES, LANES)] = zeros

    lane16 = lax.iota(jnp.int32, 16) * BSLOTS
    ones = jnp.ones((LANES,), jnp.float32)
    b9 = jnp.full((LANES,), _B9, jnp.float32)

    def vec_body(i, _):
        conf = conf_v[pl.ds(i * 16, 16)]
        pred = pred_v[pl.ds(i * 16, 16)]
        targ = targ_v[pl.ds(i * 16, 16)]
        accv = jnp.where(pred == targ, 1.0, 0.0).astype(jnp.float32)
        b0 = jnp.clip((conf * 10.0).astype(jnp.int32), 0, 9)
        b0f = b0.astype(jnp.float32)
        lof = jnp.where(b0 == 9, b9, b0f * 0.1)
        hif = jnp.where(b0 == 8, b9, (b0f + 1.0) * 0.1)
        one = jnp.ones((LANES,), jnp.int32)
        zero = jnp.zeros((LANES,), jnp.int32)
        b1 = b0 + jnp.where(conf > hif, one, zero) - jnp.where(conf <= lof, one, zero)
        addr = lane16 + (b1 & 15)
        plsc.addupdate_scatter(acc_v, [addr], ones)
        plsc.addupdate_scatter(acc_v, [addr + SEG], conf)
        plsc.addupdate_scatter(acc_v, [addr + 2 * SEG], accv)
        return 0

    def chunk_body(j, _):
        c = wid + NW * j

        @pl.when(c < NCHUNKS)
        def _():
            start = c * CHUNK
            pltpu.sync_copy(conf_hbm.at[pl.ds(start, CHUNK)], conf_v)
            pltpu.sync_copy(pred_hbm.at[pl.ds(start, CHUNK)], pred_v)
            pltpu.sync_copy(targ_hbm.at[pl.ds(start, CHUNK)], targ_v)
            lax.fori_loop(0, VECS, vec_body, 0)

        return 0

    lax.fori_loop(0, JMAX, chunk_body, 0)
    pltpu.sync_copy(acc_v, out_hbm.at[wid])


_mesh = plsc.VectorSubcoreMesh(core_axis_name="c", subcore_axis_name="s")

_ece_partials = functools.partial(
    pl.kernel,
    out_type=jax.ShapeDtypeStruct((NW, ACCLEN), jnp.float32),
    mesh=_mesh,
    scratch_types=[
        pltpu.VMEM((CHUNK,), jnp.float32),
        pltpu.VMEM((CHUNK,), jnp.int32),
        pltpu.VMEM((CHUNK,), jnp.int32),
        pltpu.VMEM((ACCLEN,), jnp.float32),
    ],
    compiler_params=pltpu.CompilerParams(needs_layout_passes=False),
)(_tec_body)


@jax.jit
def kernel(confidences, predictions, targets):
    parts = _ece_partials(confidences, predictions, targets)
    sums = parts.reshape(NW, 3, LANES, BSLOTS)[..., :NBINS].sum(axis=(0, 2))
    cnt, s_conf, s_acc = sums[0], sums[1], sums[2]
    safe = jnp.maximum(cnt, 1.0)
    contrib = (cnt / N) * jnp.abs(s_acc / safe - s_conf / safe)
    ece = jnp.sum(jnp.where(cnt > 0.0, contrib, 0.0))
    return jnp.float32(0.1) * ece


# speedup vs baseline: 2.0072x; 2.0072x over previous
import functools

import jax
import jax.numpy as jnp
from jax import lax
from jax.experimental import pallas as pl
from jax.experimental.pallas import tpu as pltpu
from jax.experimental.pallas import tpu_sc as plsc

N = 2_000_000
CHUNK = 8_000
NCHUNKS = N // CHUNK
VECS = CHUNK // 16
NW = 32
JMAX = -(-NCHUNKS // NW)
NBINS = 10
LANES = 16
BSLOTS = 16
SEG = BSLOTS * LANES
ACCLEN = 2 * SEG
PACK = 4096.0

_B9 = 0.9


def _tec_body(conf_hbm, pred_hbm, targ_hbm, out_hbm, conf_v, pred_v, targ_v, acc_v):
    wid = lax.axis_index("s") * 2 + lax.axis_index("c")

    zeros = jnp.zeros((LANES,), jnp.float32)
    for i in range(ACCLEN // LANES):
        acc_v[pl.ds(i * LANES, LANES)] = zeros

    lane = lax.iota(jnp.int32, 16)
    b9 = jnp.full((LANES,), _B9, jnp.float32)
    one = jnp.ones((LANES,), jnp.int32)
    zero = jnp.zeros((LANES,), jnp.int32)

    def vec_body(i):
        conf = conf_v[pl.ds(i * 16, 16)]
        pred = pred_v[pl.ds(i * 16, 16)]
        targ = targ_v[pl.ds(i * 16, 16)]
        packed = jnp.where(pred == targ, 1.0 + PACK, 1.0).astype(jnp.float32)
        b0 = jnp.minimum((conf * 10.0).astype(jnp.int32), 9)
        b0f = b0.astype(jnp.float32)
        lof = jnp.where(b0 == 9, b9, b0f * 0.1)
        hif = jnp.where(b0 == 8, b9, (b0f + 1.0) * 0.1)
        b1 = b0 + jnp.where(conf > hif, one, zero) - jnp.where(conf <= lof, one, zero)
        addr = (b1 & 15) * 16 + lane
        plsc.addupdate_scatter(acc_v, [addr], packed)
        plsc.addupdate_scatter(acc_v, [addr + SEG], conf)

    def chunk_body(j, _):
        c = wid + NW * j

        @pl.when(c < NCHUNKS)
        def _():
            start = c * CHUNK
            pltpu.sync_copy(conf_hbm.at[pl.ds(start, CHUNK)], conf_v)
            pltpu.sync_copy(pred_hbm.at[pl.ds(start, CHUNK)], pred_v)
            pltpu.sync_copy(targ_hbm.at[pl.ds(start, CHUNK)], targ_v)
            plsc.parallel_loop(0, VECS, unroll=4)(vec_body)

        return 0

    lax.fori_loop(0, JMAX, chunk_body, 0)
    pltpu.sync_copy(acc_v, out_hbm.at[wid])


_mesh = plsc.VectorSubcoreMesh(core_axis_name="c", subcore_axis_name="s")

_ece_partials = functools.partial(
    pl.kernel,
    out_type=jax.ShapeDtypeStruct((NW, ACCLEN), jnp.float32),
    mesh=_mesh,
    scratch_types=[
        pltpu.VMEM((CHUNK,), jnp.float32),
        pltpu.VMEM((CHUNK,), jnp.int32),
        pltpu.VMEM((CHUNK,), jnp.int32),
        pltpu.VMEM((ACCLEN,), jnp.float32),
    ],
    compiler_params=pltpu.CompilerParams(needs_layout_passes=False),
)(_tec_body)


@jax.jit
def kernel(confidences, predictions, targets):
    parts = _ece_partials(confidences, predictions, targets)
    view = parts.reshape(NW, 2, BSLOTS, LANES)[:, :, :NBINS, :]
    packed = view[:, 0]
    s_conf = view[:, 1].sum(axis=(0, 2))
    acc = jnp.floor(packed / PACK)
    cnt = (packed - acc * PACK).sum(axis=(0, 2))
    s_acc = acc.sum(axis=(0, 2))
    safe = jnp.maximum(cnt, 1.0)
    contrib = (cnt / N) * jnp.abs(s_acc / safe - s_conf / safe)
    ece = jnp.sum(jnp.where(cnt > 0.0, contrib, 0.0))
    return jnp.float32(0.1) * ece


# speedup vs baseline: 2.7802x; 1.3851x over previous
import functools

import jax
import jax.numpy as jnp
from jax import lax
from jax.experimental import pallas as pl
from jax.experimental.pallas import tpu as pltpu
from jax.experimental.pallas import tpu_sc as plsc

N = 2_000_000
CHUNK = 8_000
NCHUNKS = N // CHUNK
VECS = CHUNK // 16
NW = 32
JMAX = -(-NCHUNKS // NW)
NBINS = 10
LANES = 16
BSLOTS = 16
SEG = BSLOTS * LANES
ACCLEN = 2 * SEG
PACK = 4096.0

_B9 = 0.9


def _tec_body(conf_hbm, pred_hbm, targ_hbm, out_hbm,
              conf_b0, pred_b0, targ_b0, conf_b1, pred_b1, targ_b1,
              acc_v, sem0, sem1):
    wid = lax.axis_index("s") * 2 + lax.axis_index("c")
    bufs = ((conf_b0, pred_b0, targ_b0, sem0), (conf_b1, pred_b1, targ_b1, sem1))

    zeros = jnp.zeros((LANES,), jnp.float32)
    for i in range(ACCLEN // LANES):
        acc_v[pl.ds(i * LANES, LANES)] = zeros

    lane = lax.iota(jnp.int32, 16)
    b9 = jnp.full((LANES,), _B9, jnp.float32)
    one = jnp.ones((LANES,), jnp.int32)
    zero = jnp.zeros((LANES,), jnp.int32)

    def make_vec_body(conf_v, pred_v, targ_v):
        def vec_body(i):
            conf = conf_v[pl.ds(i * 16, 16)]
            pred = pred_v[pl.ds(i * 16, 16)]
            targ = targ_v[pl.ds(i * 16, 16)]
            packed = jnp.where(pred == targ, 1.0 + PACK, 1.0).astype(jnp.float32)
            b0 = jnp.minimum((conf * 10.0).astype(jnp.int32), 9)
            b0f = b0.astype(jnp.float32)
            lof = jnp.where(b0 == 9, b9, b0f * 0.1)
            hif = jnp.where(b0 == 8, b9, (b0f + 1.0) * 0.1)
            b1 = b0 + jnp.where(conf > hif, one, zero) - jnp.where(conf <= lof, one, zero)
            addr = (b1 & 15) * 16 + lane
            plsc.addupdate_scatter(acc_v, [addr], packed)
            plsc.addupdate_scatter(acc_v, [addr + SEG], conf)
        return vec_body

    def fire(j):
        c = wid + NW * j
        cv, pv, tv, sem = bufs[j % 2]

        @pl.when(c < NCHUNKS)
        def _():
            s = c * CHUNK
            pltpu.async_copy(conf_hbm.at[pl.ds(s, CHUNK)], cv, sem)
            pltpu.async_copy(pred_hbm.at[pl.ds(s, CHUNK)], pv, sem)
            pltpu.async_copy(targ_hbm.at[pl.ds(s, CHUNK)], tv, sem)

    def drain_and_process(j):
        c = wid + NW * j
        cv, pv, tv, sem = bufs[j % 2]

        @pl.when(c < NCHUNKS)
        def _():
            s = c * CHUNK
            pltpu.make_async_copy(conf_hbm.at[pl.ds(s, CHUNK)], cv, sem).wait()
            pltpu.make_async_copy(pred_hbm.at[pl.ds(s, CHUNK)], pv, sem).wait()
            pltpu.make_async_copy(targ_hbm.at[pl.ds(s, CHUNK)], tv, sem).wait()
            plsc.parallel_loop(0, VECS, unroll=4)(make_vec_body(cv, pv, tv))

    fire(0)
    for j in range(JMAX):
        if j + 1 < JMAX:
            fire(j + 1)
        drain_and_process(j)

    pltpu.sync_copy(acc_v, out_hbm.at[wid])


_mesh = plsc.VectorSubcoreMesh(core_axis_name="c", subcore_axis_name="s")

_ece_partials = functools.partial(
    pl.kernel,
    out_type=jax.ShapeDtypeStruct((NW, ACCLEN), jnp.float32),
    mesh=_mesh,
    scratch_types=[
        pltpu.VMEM((CHUNK,), jnp.float32),
        pltpu.VMEM((CHUNK,), jnp.int32),
        pltpu.VMEM((CHUNK,), jnp.int32),
        pltpu.VMEM((CHUNK,), jnp.float32),
        pltpu.VMEM((CHUNK,), jnp.int32),
        pltpu.VMEM((CHUNK,), jnp.int32),
        pltpu.VMEM((ACCLEN,), jnp.float32),
        pltpu.SemaphoreType.DMA,
        pltpu.SemaphoreType.DMA,
    ],
    compiler_params=pltpu.CompilerParams(needs_layout_passes=False),
)(_tec_body)


@jax.jit
def kernel(confidences, predictions, targets):
    parts = _ece_partials(confidences, predictions, targets)
    view = parts.reshape(NW, 2, BSLOTS, LANES)[:, :, :NBINS, :]
    packed = view[:, 0]
    s_conf = view[:, 1].sum(axis=(0, 2))
    acc = jnp.floor(packed / PACK)
    cnt = (packed - acc * PACK).sum(axis=(0, 2))
    s_acc = acc.sum(axis=(0, 2))
    safe = jnp.maximum(cnt, 1.0)
    contrib = (cnt / N) * jnp.abs(s_acc / safe - s_conf / safe)
    ece = jnp.sum(jnp.where(cnt > 0.0, contrib, 0.0))
    return jnp.float32(0.1) * ece


# speedup vs baseline: 3.1080x; 1.1179x over previous
import functools

import jax
import jax.numpy as jnp
from jax import lax
from jax.experimental import pallas as pl
from jax.experimental.pallas import tpu as pltpu
from jax.experimental.pallas import tpu_sc as plsc

N = 2_000_000
CHUNK = 8_000
NCHUNKS = N // CHUNK
VECS = CHUNK // 16
NW = 32
JMAX = -(-NCHUNKS // NW)
NBINS = 10
LANES = 16
BSLOTS = 16
SEG = BSLOTS * LANES
ACCLEN = 2 * SEG
PACK = 4096.0


def _tec_body(conf_hbm, pred_hbm, targ_hbm, bnds_hbm, out_hbm,
              conf_b0, pred_b0, targ_b0, conf_b1, pred_b1, targ_b1,
              bnd_v, acc_v, sem0, sem1):
    wid = lax.axis_index("s") * 2 + lax.axis_index("c")
    bufs = ((conf_b0, pred_b0, targ_b0, sem0), (conf_b1, pred_b1, targ_b1, sem1))

    pltpu.sync_copy(bnds_hbm, bnd_v)
    zeros = jnp.zeros((LANES,), jnp.float32)
    for i in range(ACCLEN // LANES):
        acc_v[pl.ds(i * LANES, LANES)] = zeros

    lane = lax.iota(jnp.int32, 16)
    dump = lane + (BSLOTS - 1) * 16
    one = jnp.ones((LANES,), jnp.int32)
    zero = jnp.zeros((LANES,), jnp.int32)

    def make_vec_body(conf_v, pred_v, targ_v):
        def vec_body(i):
            conf = conf_v[pl.ds(i * 16, 16)]
            pred = pred_v[pl.ds(i * 16, 16)]
            targ = targ_v[pl.ds(i * 16, 16)]
            packed = jnp.where(pred == targ, 1.0 + PACK, 1.0).astype(jnp.float32)
            t = (conf * 10.0 - 0.5).astype(jnp.int32)
            hi = plsc.load_gather(bnd_v, [t + one])
            b1 = t + jnp.where(conf > hi, one, zero)
            addr = jnp.where(conf > 0.0, b1 * 16 + lane, dump)
            plsc.addupdate_scatter(acc_v, [addr], packed)
            plsc.addupdate_scatter(acc_v, [addr + SEG], conf)
        return vec_body

    def fire(j):
        c = wid + NW * j
        cv, pv, tv, sem = bufs[j % 2]

        @pl.when(c < NCHUNKS)
        def _():
            s = c * CHUNK
            pltpu.async_copy(conf_hbm.at[pl.ds(s, CHUNK)], cv, sem)
            pltpu.async_copy(pred_hbm.at[pl.ds(s, CHUNK)], pv, sem)
            pltpu.async_copy(targ_hbm.at[pl.ds(s, CHUNK)], tv, sem)

    def drain_and_process(j):
        c = wid + NW * j
        cv, pv, tv, sem = bufs[j % 2]

        @pl.when(c < NCHUNKS)
        def _():
            s = c * CHUNK
            pltpu.make_async_copy(conf_hbm.at[pl.ds(s, CHUNK)], cv, sem).wait()
            pltpu.make_async_copy(pred_hbm.at[pl.ds(s, CHUNK)], pv, sem).wait()
            pltpu.make_async_copy(targ_hbm.at[pl.ds(s, CHUNK)], tv, sem).wait()
            plsc.parallel_loop(0, VECS, unroll=4)(make_vec_body(cv, pv, tv))

    fire(0)
    for j in range(JMAX):
        if j + 1 < JMAX:
            fire(j + 1)
        drain_and_process(j)

    pltpu.sync_copy(acc_v, out_hbm.at[wid])


_mesh = plsc.VectorSubcoreMesh(core_axis_name="c", subcore_axis_name="s")

_ece_partials = functools.partial(
    pl.kernel,
    out_type=jax.ShapeDtypeStruct((NW, ACCLEN), jnp.float32),
    mesh=_mesh,
    scratch_types=[
        pltpu.VMEM((CHUNK,), jnp.float32),
        pltpu.VMEM((CHUNK,), jnp.int32),
        pltpu.VMEM((CHUNK,), jnp.int32),
        pltpu.VMEM((CHUNK,), jnp.float32),
        pltpu.VMEM((CHUNK,), jnp.int32),
        pltpu.VMEM((CHUNK,), jnp.int32),
        pltpu.VMEM((LANES,), jnp.float32),
        pltpu.VMEM((ACCLEN,), jnp.float32),
        pltpu.SemaphoreType.DMA,
        pltpu.SemaphoreType.DMA,
    ],
    compiler_params=pltpu.CompilerParams(needs_layout_passes=False),
)(_tec_body)


import numpy as np

_BNDS = jnp.asarray(
    np.pad(np.linspace(0.0, 1.0, NBINS + 1).astype(np.float32), (0, 5)))


@jax.jit
def kernel(confidences, predictions, targets):
    parts = _ece_partials(confidences, predictions, targets, _BNDS)
    view = parts.reshape(NW, 2, BSLOTS, LANES)[:, :, :NBINS, :]
    packed = view[:, 0]
    s_conf = view[:, 1].sum(axis=(0, 2))
    acc = jnp.floor(packed / PACK)
    cnt = (packed - acc * PACK).sum(axis=(0, 2))
    s_acc = acc.sum(axis=(0, 2))
    safe = jnp.maximum(cnt, 1.0)
    contrib = (cnt / N) * jnp.abs(s_acc / safe - s_conf / safe)
    ece = jnp.sum(jnp.where(cnt > 0.0, contrib, 0.0))
    return jnp.float32(0.1) * ece
